# baseline (speedup 1.0000x reference)
import jax
import jax.numpy as jnp
from jax import lax
from jax.experimental import pallas as pl
from jax.experimental.pallas import tpu as pltpu
from jax.experimental.pallas import tpu_sc as plsc

_H = 16
_S = 4096
_NC = 2
_NS = 16
_NW = _NC * _NS
_ROWS_PER_W = _H * _S // _NW
_BLOCKS_PER_W = _ROWS_PER_W // 8
_LAG = 4


def _sc_body(shifts_hbm, out_hbm, buf, sem):
    wid = lax.axis_index("s") * _NC + lax.axis_index("c")
    h = wid // 2
    half = wid % 2
    pltpu.sync_copy(shifts_hbm.at[h], buf)

    row_base = half * _ROWS_PER_W
    q_base = 511 - (_BLOCKS_PER_W * half)

    drain_one = pltpu.make_async_copy(
        buf.at[:, pl.ds(0, _S)], out_hbm.at[0, 0, pl.ds(0, 8), :], sem)

    def body(b, _):
        q = q_base - b
        src = buf.at[:, pl.ds(pl.multiple_of(8 * q, 8), _S)]
        dst = out_hbm.at[
            0, h, pl.ds(pl.multiple_of(row_base + 8 * b, 8), 8), :]
        pltpu.async_copy(src, dst, sem)

        @pl.when(b >= _LAG)
        def _():
            drain_one.wait()
        return 0

    lax.fori_loop(0, _BLOCKS_PER_W, body, 0)
    for _ in range(_LAG):
        drain_one.wait()


@jax.jit
def _expand(shifts):
    mesh = plsc.VectorSubcoreMesh(core_axis_name="c", subcore_axis_name="s")
    return pl.kernel(
        _sc_body,
        out_type=jax.ShapeDtypeStruct((1, _H, _S, _S), jnp.float32),
        mesh=mesh,
        scratch_types=[
            pltpu.VMEM((8, 8192), jnp.float32),
            pltpu.SemaphoreType.DMA,
        ],
        compiler_params=pltpu.CompilerParams(use_tc_tiling_on_sc=False),
    )(shifts)


def kernel(qlen, klen, relative_attention_bias):
    tt = relative_attention_bias.T
    ttp = jnp.pad(tt, ((0, 0), (0, 8192 + 7 - tt.shape[1])))
    shifts = jnp.stack([ttp[:, 7 - s:7 - s + 8192] for s in range(8)], axis=1)
    return _expand(shifts)


# speedup vs baseline: 2.7323x; 2.7323x over previous
import jax
import jax.numpy as jnp
from jax import lax
from jax.experimental import pallas as pl
from jax.experimental.pallas import tpu as pltpu
from jax.experimental.pallas import tpu_sc as plsc

_H = 16
_S = 4096
_NC = 2
_NS = 16
_NW = _NC * _NS
_K = 8064
_UNITS_PER_W = _H * 8 // _NW
_BLOCKS_PER_U = 32
_LAG = 4


def _sc_body(shifts_hbm, out_hbm, buf, sem):
    wid = lax.axis_index("s") * _NC + lax.axis_index("c")

    drain_store = pltpu.make_async_copy(
        buf.at[:, pl.ds(0, _S)], out_hbm.at[0, 0, pl.ds(0, 16), :], sem)

    def unit(c, _):
        u = wid * _UNITS_PER_W + c
        h = u // 8
        t = u % 8

        @pl.when(c > 0)
        def _():
            for _ in range(_LAG):
                drain_store.wait()

        pltpu.sync_copy(shifts_hbm.at[h, t], buf)

        def body(qq, _):
            i0 = 16 * (255 - t - 8 * qq)
            src = buf.at[:, pl.ds(pl.multiple_of(128 * qq, 128), _S)]
            dst = out_hbm.at[0, h, pl.ds(pl.multiple_of(i0, 16), 16), :]
            pltpu.async_copy(src, dst, sem)

            @pl.when(qq >= _LAG)
            def _():
                drain_store.wait()
            return 0

        lax.fori_loop(0, _BLOCKS_PER_U, body, 0)
        return 0

    lax.fori_loop(0, _UNITS_PER_W, unit, 0)
    for _ in range(_LAG):
        drain_store.wait()


@jax.jit
def _expand(shifts):
    mesh = plsc.VectorSubcoreMesh(core_axis_name="c", subcore_axis_name="s")
    return pl.kernel(
        _sc_body,
        out_type=jax.ShapeDtypeStruct((1, _H, _S, _S), jnp.float32),
        mesh=mesh,
        scratch_types=[
            pltpu.VMEM((16, _K), jnp.float32),
            pltpu.SemaphoreType.DMA,
        ],
        compiler_params=pltpu.CompilerParams(use_tc_tiling_on_sc=True),
    )(shifts)


def kernel(qlen, klen, relative_attention_bias):
    tt = relative_attention_bias.T
    shifts = jnp.stack(
        [jnp.stack([tt[:, 16 * t + 15 - d:16 * t + 15 - d + _K]
                    for d in range(16)], axis=1)
         for t in range(8)], axis=1)
    return _expand(shifts)


# speedup vs baseline: 2.7771x; 1.0164x over previous
import jax
import jax.numpy as jnp
from jax import lax
from jax.experimental import pallas as pl
from jax.experimental.pallas import tpu as pltpu
from jax.experimental.pallas import tpu_sc as plsc

_H = 16
_S = 4096
_NC = 2
_NS = 16
_NW = _NC * _NS
_K = 8064
_UNITS_PER_W = _H * 16 // _NW
_BLOCKS_PER_U = 32
_LAG = 8


def _sc_body(shifts_hbm, out_hbm, buf0, buf1, sem, lsem):
    wid = lax.axis_index("s") * _NC + lax.axis_index("c")
    bufs = (buf0, buf1)

    drain_store = pltpu.make_async_copy(
        buf0.at[:, pl.ds(0, _S)], out_hbm.at[0, 0, pl.ds(0, 8), :], sem)
    drain_load = pltpu.make_async_copy(shifts_hbm.at[0, 0], buf0, lsem)

    def ht(c):
        u = wid * _UNITS_PER_W + c
        return u // 16, u % 16

    h0, t0 = ht(0)
    pltpu.async_copy(shifts_hbm.at[h0, t0], buf0, lsem)

    for c in range(_UNITS_PER_W):
        h, t = ht(c)
        buf = bufs[c % 2]

        if c > 0:
            for _ in range(_LAG):
                drain_store.wait()
        drain_load.wait()
        if c + 1 < _UNITS_PER_W:
            hn, tn = ht(c + 1)
            pltpu.async_copy(shifts_hbm.at[hn, tn], bufs[(c + 1) % 2], lsem)

        def body(qq, _, buf=buf, h=h, t=t):
            i0 = 4088 - 128 * qq - 8 * t
            src = buf.at[:, pl.ds(pl.multiple_of(128 * qq, 128), _S)]
            dst = out_hbm.at[0, h, pl.ds(pl.multiple_of(i0, 8), 8), :]
            pltpu.async_copy(src, dst, sem)

            @pl.when(qq >= _LAG)
            def _():
                drain_store.wait()
            return 0

        lax.fori_loop(0, _BLOCKS_PER_U, body, 0)

    for _ in range(_LAG):
        drain_store.wait()


@jax.jit
def _expand(shifts):
    mesh = plsc.VectorSubcoreMesh(core_axis_name="c", subcore_axis_name="s")
    return pl.kernel(
        _sc_body,
        out_type=jax.ShapeDtypeStruct((1, _H, _S, _S), jnp.float32),
        mesh=mesh,
        scratch_types=[
            pltpu.VMEM((8, _K), jnp.float32),
            pltpu.VMEM((8, _K), jnp.float32),
            pltpu.SemaphoreType.DMA,
            pltpu.SemaphoreType.DMA,
        ],
        compiler_params=pltpu.CompilerParams(use_tc_tiling_on_sc=True),
    )(shifts)


def kernel(qlen, klen, relative_attention_bias):
    tt = relative_attention_bias.T
    shifts = jnp.stack(
        [jnp.stack([tt[:, 8 * t + 7 - s:8 * t + 7 - s + _K]
                    for s in range(8)], axis=1)
         for t in range(16)], axis=1)
    return _expand(shifts)
